# baseline (speedup 1.0000x reference)
import functools

import jax
import jax.numpy as jnp
from jax import lax
from jax.experimental import pallas as pl
from jax.experimental.pallas import tpu as pltpu
from jax.experimental.pallas import tpu_sc as plsc

MAP = 500
CMEM = 32
H_IN, W_IN = 128, 256
H_RS, W_RS = 256, 512
V = H_RS * W_RS
B_PAD = 262144
WPAD = 640
HPAD = 520
ROWB = 8
NROW_OUT = 512
N_STAT = MAP * MAP


def _resize_body(r_ref, x_ref, ct_ref, o_ref):
    R = r_ref[...]
    CT = ct_ref[...]
    for c in range(CMEM):
        t = lax.dot(R, x_ref[c], preferred_element_type=jnp.float32)
        o_ref[c] = lax.dot(t, CT, preferred_element_type=jnp.float32)


def _interp_matrix(n_out, n_in, full_out):
    pos = (jnp.arange(n_out, dtype=jnp.float32) * 4.0) * (
        (n_in - 1.0) / (full_out - 1.0))
    i0 = jnp.floor(pos).astype(jnp.int32)
    i1 = jnp.minimum(i0 + 1, n_in - 1)
    w = pos - i0.astype(jnp.float32)
    cols = jnp.arange(n_in, dtype=jnp.int32)[None, :]
    m = (cols == i0[:, None]) * (1.0 - w[:, None]) + (cols == i1[:, None]) * w[:, None]
    same = (i0 == i1)[:, None]
    m = jnp.where(same & (cols == i0[:, None]), 1.0, m)
    return m.astype(jnp.float32)


def _resize(features):
    x = features[0, 0]
    R = _interp_matrix(H_RS, H_IN, 1024)
    CT = _interp_matrix(W_RS, W_IN, 2048).T
    out = pl.pallas_call(
        _resize_body,
        out_shape=jax.ShapeDtypeStruct((CMEM, H_RS, W_RS), jnp.float32),
    )(R, x, CT)
    return out


def _mask_body(p_ref, m_ref, idx_ref, smax):
    step = pl.program_id(0)

    @pl.when(step == 0)
    def _():
        smax[0] = jnp.max(p_ref[...])

    @pl.when(step == 1)
    def _():
        p = p_ref[...]
        m = p < smax[0]
        m_ref[...] = m.astype(jnp.int32)
        idx_ref[...] = jnp.where(m, p, V)


def _mask_and_indices(proj):
    p2 = proj.reshape(MAP, MAP)
    m, idx = pl.pallas_call(
        _mask_body,
        grid=(2,),
        in_specs=[pl.BlockSpec((MAP, MAP), lambda i: (0, 0))],
        out_specs=[pl.BlockSpec((MAP, MAP), lambda i: (0, 0)),
                   pl.BlockSpec((MAP, MAP), lambda i: (0, 0))],
        out_shape=[jax.ShapeDtypeStruct((MAP, MAP), jnp.int32),
                   jax.ShapeDtypeStruct((MAP, MAP), jnp.int32)],
        scratch_shapes=[pltpu.SMEM((1,), jnp.int32)],
    )(p2)
    return m, idx


_CH = 512


def _sc_gather(table, idx_flat):
    info = plsc.get_sparse_core_info()
    nw = info.num_cores * info.num_subcores
    b_per_w = B_PAD // nw
    nchunk = b_per_w // _CH
    mesh = plsc.VectorSubcoreMesh(core_axis_name="c", subcore_axis_name="s")

    @functools.partial(
        pl.kernel,
        mesh=mesh,
        out_type=jax.ShapeDtypeStruct((B_PAD, 128), jnp.float32),
        scratch_types=[
            pltpu.VMEM((_CH,), jnp.int32),
            pltpu.VMEM((_CH, 128), jnp.float32),
            pltpu.SemaphoreType.DMA,
        ],
    )
    def k(table_hbm, idx_hbm, out_hbm, idx_v, rows_v, sem):
        wid = lax.axis_index("s") * info.num_cores + lax.axis_index("c")
        base = wid * b_per_w
        for j in range(nchunk):
            off = base + j * _CH
            pltpu.sync_copy(idx_hbm.at[pl.ds(off, _CH)], idx_v)
            pltpu.async_copy(table_hbm.at[idx_v], rows_v, sem).wait()
            pltpu.sync_copy(rows_v, out_hbm.at[pl.ds(off, _CH)])

    return k(table, idx_flat)


def _make_conv(cin, cout, p, apply_act, with_stats, with_bias):
    K = 2 * p + 1
    nblk = NROW_OUT // ROWB

    def body(*refs):
        if with_bias:
            a_ref, d_ref, w_ref, bias_ref = refs[:4]
            xrefs = refs[4:4 + (2 if p > 0 else 1)]
            orefs = refs[4 + (2 if p > 0 else 1):]
        else:
            a_ref, d_ref, w_ref = refs[:3]
            xrefs = refs[3:3 + (2 if p > 0 else 1)]
            orefs = refs[3 + (2 if p > 0 else 1):]
        o_ref = orefs[0]

        i = pl.program_id(0)
        if p > 0:
            x = jnp.concatenate([xrefs[0][...], xrefs[1][...]], axis=1)
        else:
            x = xrefs[0][...]
        nrows = x.shape[1]

        a = a_ref[...][:, :, None]
        d = d_ref[...][:, :, None]
        xn = x * a + d
        if apply_act:
            xn = jnp.maximum(xn, 0.0)
        r0 = i * ROWB
        rows_g = r0 + lax.broadcasted_iota(jnp.int32, (1, nrows, WPAD), 1)
        cols_g = lax.broadcasted_iota(jnp.int32, (1, nrows, WPAD), 2)
        valid = ((rows_g >= p) & (rows_g < MAP + p)
                 & (cols_g >= p) & (cols_g < MAP + p))
        xn = jnp.where(valid, xn, 0.0)

        acc = jnp.zeros((cout, ROWB, 512), jnp.float32)
        for dy in range(K):
            for dx in range(K):
                xs = lax.slice(xn, (0, dy, dx), (cin, dy + ROWB, dx + 512))
                wt = w_ref[:, :, dy, dx]
                acc = acc + lax.dot_general(
                    wt, xs, (((1,), (0,)), ((), ())),
                    preferred_element_type=jnp.float32)
        if with_bias:
            acc = acc + bias_ref[...][:, :, None]
        o_ref[...] = acc

        if with_stats:
            s_ref, ss_ref = orefs[1], orefs[2]
            ro = r0 + lax.broadcasted_iota(jnp.int32, (1, ROWB, 1), 1)
            val = jnp.where(ro < MAP, acc[:, :, :MAP], 0.0)
            ps = jnp.sum(val, axis=(1, 2))[:, None]
            pss = jnp.sum(val * val, axis=(1, 2))[:, None]

            @pl.when(i == 0)
            def _():
                s_ref[...] = ps
                ss_ref[...] = pss

            @pl.when(i > 0)
            def _():
                s_ref[...] += ps
                ss_ref[...] += pss

    in_specs = [
        pl.BlockSpec((cin, 1), lambda i: (0, 0)),
        pl.BlockSpec((cin, 1), lambda i: (0, 0)),
        pl.BlockSpec((cout, cin, K, K), lambda i: (0, 0, 0, 0)),
    ]
    if with_bias:
        in_specs.append(pl.BlockSpec((cout, 1), lambda i: (0, 0)))
    in_specs.append(pl.BlockSpec((cin, ROWB, WPAD), lambda i: (0, i, 0)))
    if p > 0:
        in_specs.append(pl.BlockSpec((cin, ROWB, WPAD), lambda i: (0, i + 1, 0)))

    out_specs = [pl.BlockSpec((cout, ROWB, 512), lambda i: (0, i, 0))]
    out_shape = [jax.ShapeDtypeStruct((cout, NROW_OUT, 512), jnp.float32)]
    if with_stats:
        out_specs += [pl.BlockSpec((cout, 1), lambda i: (0, 0)),
                      pl.BlockSpec((cout, 1), lambda i: (0, 0))]
        out_shape += [jax.ShapeDtypeStruct((cout, 1), jnp.float32),
                      jax.ShapeDtypeStruct((cout, 1), jnp.float32)]

    call = pl.pallas_call(
        body, grid=(nblk,),
        in_specs=in_specs, out_specs=out_specs, out_shape=out_shape,
    )
    if p > 0:
        return lambda *args: call(*args, args[-1])
    return call


def _pad_map(y, p):
    yv = y[:, :MAP, :MAP]
    return jnp.pad(yv, ((0, 0), (p, HPAD - MAP - p), (p, WPAD - MAP - p)))


def _bn_coeffs(s, ss, g, b):
    mu = s[:, 0] / N_STAT
    var = ss[:, 0] / N_STAT - mu * mu
    a = g / jnp.sqrt(var + 1e-5)
    d = b - mu * a
    return a[:, None], d[:, None]


def kernel(features, proj_indices, masks_inliers, rgb_no_norm,
           W1, g1, b1, W2, g2, b2, W3, g3, b3, W4, g4, b4, W5, b5):
    del masks_inliers, rgb_no_norm

    rs = _resize(features)
    table = jnp.transpose(rs, (1, 2, 0)).reshape(V, CMEM)
    table = jnp.pad(table, ((0, 1), (0, 128 - CMEM)))

    m, idx = _mask_and_indices(proj_indices)
    idx_flat = jnp.pad(idx.reshape(-1), (0, B_PAD - MAP * MAP),
                       constant_values=V)

    gathered = _sc_gather(table, idx_flat)
    mem = jnp.transpose(gathered[:MAP * MAP, :CMEM], (1, 0)).reshape(
        CMEM, MAP, MAP)

    ones32 = jnp.ones((CMEM, 1), jnp.float32)
    zeros32 = jnp.zeros((CMEM, 1), jnp.float32)

    conv1 = _make_conv(32, 128, 3, False, True, False)
    y1, s1, ss1 = conv1(ones32, zeros32, W1, _pad_map(mem, 3))

    a1, d1 = _bn_coeffs(s1, ss1, g1, b1)
    conv2 = _make_conv(128, 64, 1, True, True, False)
    y2, s2, ss2 = conv2(a1, d1, W2, _pad_map(y1, 1))

    a2, d2 = _bn_coeffs(s2, ss2, g2, b2)
    conv3 = _make_conv(64, 48, 1, True, True, False)
    y3, s3, ss3 = conv3(a2, d2, W3, _pad_map(y2, 1))

    a3, d3 = _bn_coeffs(s3, ss3, g3, b3)
    conv4 = _make_conv(48, 48, 1, True, True, False)
    y4, s4, ss4 = conv4(a3, d3, W4, _pad_map(y3, 1))

    a4, d4 = _bn_coeffs(s4, ss4, g4, b4)
    conv5 = _make_conv(48, 21, 0, True, False, True)
    (y5,) = conv5(a4, d4, W5, b5[:, None], _pad_map(y4, 0))

    semmap = y5[:, :MAP, :MAP].reshape(1, 21, MAP, MAP)
    observed = (m > 0).reshape(1, MAP, MAP)
    return (semmap, observed)


# speedup vs baseline: 1.0014x; 1.0014x over previous
import functools

import jax
import jax.numpy as jnp
from jax import lax
from jax.experimental import pallas as pl
from jax.experimental.pallas import tpu as pltpu
from jax.experimental.pallas import tpu_sc as plsc

MAP = 500
CMEM = 32
H_IN, W_IN = 128, 256
H_RS, W_RS = 256, 512
V = H_RS * W_RS
B_PAD = 262144
WPAD = 640
HPAD = 520
ROWB = 8
NROW_OUT = 512
N_STAT = MAP * MAP


def _resize_body(r_ref, x_ref, ct_ref, o_ref):
    R = r_ref[...]
    CT = ct_ref[...]
    for c in range(CMEM):
        t = lax.dot(R, x_ref[c], preferred_element_type=jnp.float32)
        o_ref[c] = lax.dot(t, CT, preferred_element_type=jnp.float32)


def _interp_matrix(n_out, n_in, full_out):
    pos = (jnp.arange(n_out, dtype=jnp.float32) * 4.0) * (
        (n_in - 1.0) / (full_out - 1.0))
    i0 = jnp.floor(pos).astype(jnp.int32)
    i1 = jnp.minimum(i0 + 1, n_in - 1)
    w = pos - i0.astype(jnp.float32)
    cols = jnp.arange(n_in, dtype=jnp.int32)[None, :]
    m = (cols == i0[:, None]) * (1.0 - w[:, None]) + (cols == i1[:, None]) * w[:, None]
    same = (i0 == i1)[:, None]
    m = jnp.where(same & (cols == i0[:, None]), 1.0, m)
    return m.astype(jnp.float32)


def _resize(features):
    x = features[0, 0]
    R = _interp_matrix(H_RS, H_IN, 1024)
    CT = _interp_matrix(W_RS, W_IN, 2048).T
    out = pl.pallas_call(
        _resize_body,
        out_shape=jax.ShapeDtypeStruct((CMEM, H_RS, W_RS), jnp.float32),
    )(R, x, CT)
    return out


def _mask_body(p_ref, m_ref, idx_ref, smax):
    step = pl.program_id(0)

    @pl.when(step == 0)
    def _():
        smax[0] = jnp.max(p_ref[...])

    @pl.when(step == 1)
    def _():
        p = p_ref[...]
        m = p < smax[0]
        m_ref[...] = m.astype(jnp.int32)
        idx_ref[...] = jnp.where(m, p, V)


def _mask_and_indices(proj):
    p2 = proj.reshape(MAP, MAP)
    m, idx = pl.pallas_call(
        _mask_body,
        grid=(2,),
        in_specs=[pl.BlockSpec((MAP, MAP), lambda i: (0, 0))],
        out_specs=[pl.BlockSpec((MAP, MAP), lambda i: (0, 0)),
                   pl.BlockSpec((MAP, MAP), lambda i: (0, 0))],
        out_shape=[jax.ShapeDtypeStruct((MAP, MAP), jnp.int32),
                   jax.ShapeDtypeStruct((MAP, MAP), jnp.int32)],
        scratch_shapes=[pltpu.SMEM((1,), jnp.int32)],
    )(p2)
    return m, idx


_CH = 256


def _sc_gather(table, idx_flat):
    info = plsc.get_sparse_core_info()
    nw = info.num_cores * info.num_subcores
    b_per_w = B_PAD // nw
    nouter = b_per_w // (2 * _CH)
    mesh = plsc.VectorSubcoreMesh(core_axis_name="c", subcore_axis_name="s")

    @functools.partial(
        pl.kernel,
        mesh=mesh,
        out_type=jax.ShapeDtypeStruct((B_PAD, 128), jnp.float32),
        scratch_types=[
            pltpu.VMEM((_CH,), jnp.int32),
            pltpu.VMEM((_CH,), jnp.int32),
            pltpu.VMEM((_CH, 128), jnp.float32),
            pltpu.VMEM((_CH, 128), jnp.float32),
            pltpu.SemaphoreType.DMA,
        ],
    )
    def k(table_hbm, idx_hbm, out_hbm, idx_v0, idx_v1, rows_v0, rows_v1, sem):
        wid = lax.axis_index("s") * info.num_cores + lax.axis_index("c")
        base = wid * b_per_w

        def body(j, carry):
            off = base + j * (2 * _CH)
            pltpu.sync_copy(idx_hbm.at[pl.ds(off, _CH)], idx_v0)
            cp0 = pltpu.async_copy(table_hbm.at[idx_v0], rows_v0, sem)
            pltpu.sync_copy(idx_hbm.at[pl.ds(off + _CH, _CH)], idx_v1)
            cp1 = pltpu.async_copy(table_hbm.at[idx_v1], rows_v1, sem)
            cp0.wait()
            pltpu.sync_copy(rows_v0, out_hbm.at[pl.ds(off, _CH)])
            cp1.wait()
            pltpu.sync_copy(rows_v1, out_hbm.at[pl.ds(off + _CH, _CH)])
            return carry

        lax.fori_loop(0, nouter, body, 0)

    return k(table, idx_flat)


def _make_conv(cin, cout, p, apply_act, with_stats, with_bias):
    K = 2 * p + 1
    nblk = NROW_OUT // ROWB

    def body(*refs):
        if with_bias:
            a_ref, d_ref, w_ref, bias_ref = refs[:4]
            xrefs = refs[4:4 + (2 if p > 0 else 1)]
            orefs = refs[4 + (2 if p > 0 else 1):]
        else:
            a_ref, d_ref, w_ref = refs[:3]
            xrefs = refs[3:3 + (2 if p > 0 else 1)]
            orefs = refs[3 + (2 if p > 0 else 1):]
        o_ref = orefs[0]

        i = pl.program_id(0)
        if p > 0:
            x = jnp.concatenate([xrefs[0][...], xrefs[1][...]], axis=1)
        else:
            x = xrefs[0][...]
        nrows = x.shape[1]

        a = a_ref[...][:, :, None]
        d = d_ref[...][:, :, None]
        xn = x * a + d
        if apply_act:
            xn = jnp.maximum(xn, 0.0)
        r0 = i * ROWB
        rows_g = r0 + lax.broadcasted_iota(jnp.int32, (1, nrows, WPAD), 1)
        cols_g = lax.broadcasted_iota(jnp.int32, (1, nrows, WPAD), 2)
        valid = ((rows_g >= p) & (rows_g < MAP + p)
                 & (cols_g >= p) & (cols_g < MAP + p))
        xn = jnp.where(valid, xn, 0.0)

        acc = jnp.zeros((cout, ROWB, 512), jnp.float32)
        for dy in range(K):
            for dx in range(K):
                xs = lax.slice(xn, (0, dy, dx), (cin, dy + ROWB, dx + 512))
                wt = w_ref[:, :, dy, dx]
                acc = acc + lax.dot_general(
                    wt, xs, (((1,), (0,)), ((), ())),
                    preferred_element_type=jnp.float32)
        if with_bias:
            acc = acc + bias_ref[...][:, :, None]
        o_ref[...] = acc

        if with_stats:
            s_ref, ss_ref = orefs[1], orefs[2]
            ro = r0 + lax.broadcasted_iota(jnp.int32, (1, ROWB, 1), 1)
            val = jnp.where(ro < MAP, acc[:, :, :MAP], 0.0)
            ps = jnp.sum(val, axis=(1, 2))[:, None]
            pss = jnp.sum(val * val, axis=(1, 2))[:, None]

            @pl.when(i == 0)
            def _():
                s_ref[...] = ps
                ss_ref[...] = pss

            @pl.when(i > 0)
            def _():
                s_ref[...] += ps
                ss_ref[...] += pss

    in_specs = [
        pl.BlockSpec((cin, 1), lambda i: (0, 0)),
        pl.BlockSpec((cin, 1), lambda i: (0, 0)),
        pl.BlockSpec((cout, cin, K, K), lambda i: (0, 0, 0, 0)),
    ]
    if with_bias:
        in_specs.append(pl.BlockSpec((cout, 1), lambda i: (0, 0)))
    in_specs.append(pl.BlockSpec((cin, ROWB, WPAD), lambda i: (0, i, 0)))
    if p > 0:
        in_specs.append(pl.BlockSpec((cin, ROWB, WPAD), lambda i: (0, i + 1, 0)))

    out_specs = [pl.BlockSpec((cout, ROWB, 512), lambda i: (0, i, 0))]
    out_shape = [jax.ShapeDtypeStruct((cout, NROW_OUT, 512), jnp.float32)]
    if with_stats:
        out_specs += [pl.BlockSpec((cout, 1), lambda i: (0, 0)),
                      pl.BlockSpec((cout, 1), lambda i: (0, 0))]
        out_shape += [jax.ShapeDtypeStruct((cout, 1), jnp.float32),
                      jax.ShapeDtypeStruct((cout, 1), jnp.float32)]

    call = pl.pallas_call(
        body, grid=(nblk,),
        in_specs=in_specs, out_specs=out_specs, out_shape=out_shape,
    )
    if p > 0:
        return lambda *args: call(*args, args[-1])
    return call


def _pad_map(y, p):
    yv = y[:, :MAP, :MAP]
    return jnp.pad(yv, ((0, 0), (p, HPAD - MAP - p), (p, WPAD - MAP - p)))


def _bn_coeffs(s, ss, g, b):
    mu = s[:, 0] / N_STAT
    var = ss[:, 0] / N_STAT - mu * mu
    a = g / jnp.sqrt(var + 1e-5)
    d = b - mu * a
    return a[:, None], d[:, None]


def kernel(features, proj_indices, masks_inliers, rgb_no_norm,
           W1, g1, b1, W2, g2, b2, W3, g3, b3, W4, g4, b4, W5, b5):
    del masks_inliers, rgb_no_norm

    rs = _resize(features)
    table = jnp.transpose(rs, (1, 2, 0)).reshape(V, CMEM)
    table = jnp.pad(table, ((0, 1), (0, 128 - CMEM)))

    m, idx = _mask_and_indices(proj_indices)
    idx_flat = jnp.pad(idx.reshape(-1), (0, B_PAD - MAP * MAP),
                       constant_values=V)

    gathered = _sc_gather(table, idx_flat)
    mem = jnp.transpose(gathered[:MAP * MAP, :CMEM], (1, 0)).reshape(
        CMEM, MAP, MAP)

    ones32 = jnp.ones((CMEM, 1), jnp.float32)
    zeros32 = jnp.zeros((CMEM, 1), jnp.float32)

    conv1 = _make_conv(32, 128, 3, False, True, False)
    y1, s1, ss1 = conv1(ones32, zeros32, W1, _pad_map(mem, 3))

    a1, d1 = _bn_coeffs(s1, ss1, g1, b1)
    conv2 = _make_conv(128, 64, 1, True, True, False)
    y2, s2, ss2 = conv2(a1, d1, W2, _pad_map(y1, 1))

    a2, d2 = _bn_coeffs(s2, ss2, g2, b2)
    conv3 = _make_conv(64, 48, 1, True, True, False)
    y3, s3, ss3 = conv3(a2, d2, W3, _pad_map(y2, 1))

    a3, d3 = _bn_coeffs(s3, ss3, g3, b3)
    conv4 = _make_conv(48, 48, 1, True, True, False)
    y4, s4, ss4 = conv4(a3, d3, W4, _pad_map(y3, 1))

    a4, d4 = _bn_coeffs(s4, ss4, g4, b4)
    conv5 = _make_conv(48, 21, 0, True, False, True)
    (y5,) = conv5(a4, d4, W5, b5[:, None], _pad_map(y4, 0))

    semmap = y5[:, :MAP, :MAP].reshape(1, 21, MAP, MAP)
    observed = (m > 0).reshape(1, MAP, MAP)
    return (semmap, observed)
